# Initial kernel scaffold; baseline (speedup 1.0000x reference)
import functools

import jax
import jax.numpy as jnp
from jax import lax
from jax.experimental import pallas as pl
from jax.experimental.pallas import tpu as pltpu
from jax.experimental.pallas import tpu_sc as plsc

N_NODES = 10000
N_EDGES = 320000
D = 128

NW = 32
NPT = 320
NPAD = NW * NPT
E_CHUNK = 4000
NCHUNK = N_EDGES // E_CHUNK
K = 128
NEG_INF = jnp.float32(-jnp.inf)

_mesh = plsc.VectorSubcoreMesh(core_axis_name="c", subcore_axis_name="s")


@functools.partial(
    pl.kernel,
    out_type=jax.ShapeDtypeStruct((NPAD * D,), jnp.float32),
    mesh=_mesh,
    scratch_types=[
        pltpu.VMEM((NPT * D,), jnp.float32),
        pltpu.VMEM((E_CHUNK,), jnp.int32),
        pltpu.VMEM((E_CHUNK,), jnp.int32),
        pltpu.VMEM((E_CHUNK + K + 16,), jnp.int32),
        pltpu.VMEM((E_CHUNK + K + 16,), jnp.int32),
        pltpu.VMEM((K, D), jnp.float32),
        pltpu.SemaphoreType.DMA,
    ],
)
def _segmax_sc(src_hbm, dst_hbm, x_hbm, out_hbm,
               acc, sr_v, ds_v, msrc, mdst, rows, sem):
    wid = lax.axis_index("s") * 2 + lax.axis_index("c")
    lo = wid * NPT

    neg = jnp.full((16,), NEG_INF, jnp.float32)
    def init_body(i, _):
        acc[pl.ds(i * 16, 16)] = neg
        return 0
    lax.fori_loop(0, NPT * D // 16, init_body, 0)

    zero16 = jnp.zeros((16,), jnp.int32)

    def chunk_body(c, _):
        base_e = c * E_CHUNK
        pltpu.sync_copy(src_hbm.at[pl.ds(base_e, E_CHUNK)], sr_v)
        pltpu.sync_copy(dst_hbm.at[pl.ds(base_e, E_CHUNK)], ds_v)

        def filt_body(i, cnt):
            d = ds_v[pl.ds(i * 16, 16)]
            s = sr_v[pl.ds(i * 16, 16)]
            dl = d - lo
            m = (dl >= 0) & (dl < NPT)
            plsc.store_compressed(msrc.at[pl.ds(cnt, 16)], s, mask=m)
            plsc.store_compressed(mdst.at[pl.ds(cnt, 16)], dl, mask=m)
            pc = plsc.all_reduce_population_count(m)
            return cnt + pc[0]
        cnt = lax.fori_loop(0, E_CHUNK // 16, filt_body, jnp.int32(0))

        for t in range(K // 16):
            msrc[pl.ds(cnt + t * 16, 16)] = zero16

        ngroups = (cnt + (K - 1)) // K

        def group_body(g, _):
            pltpu.async_copy(x_hbm.at[msrc.at[pl.ds(g * K, K)]], rows, sem).wait()
            nloc = jnp.minimum(K, cnt - g * K)

            def edge_body(j, _):
                dl = mdst[g * K + j]
                base = dl * D
                for f in range(D // 16):
                    sl = pl.ds(base + f * 16, 16)
                    acc[sl] = jnp.maximum(acc[sl], rows[j, pl.ds(f * 16, 16)])
                return 0
            lax.fori_loop(0, nloc, edge_body, 0)
            return 0
        lax.fori_loop(0, ngroups, group_body, 0)
        return 0

    lax.fori_loop(0, NCHUNK, chunk_body, 0)

    pltpu.sync_copy(acc, out_hbm.at[pl.ds(lo * D, NPT * D)])


def _mlp_tc_body(x_ref, s_ref, w1_ref, w2_ref, b_ref, o_ref):
    x = x_ref[...]
    s = s_ref[...]
    xj = jnp.where(s == NEG_INF, 0.0, s - x)
    h = (jnp.dot(x, w1_ref[...], preferred_element_type=jnp.float32)
         + jnp.dot(xj, w2_ref[...], preferred_element_type=jnp.float32)
         + b_ref[...])
    o_ref[...] = jnp.maximum(h, 0.0)


ROWS_BLK = 500


def _mlp_tc(x, s, w1, w2, b):
    grid = (N_NODES // ROWS_BLK,)
    return pl.pallas_call(
        _mlp_tc_body,
        grid=grid,
        in_specs=[
            pl.BlockSpec((ROWS_BLK, D), lambda i: (i, 0)),
            pl.BlockSpec((ROWS_BLK, D), lambda i: (i, 0)),
            pl.BlockSpec((D, D), lambda i: (0, 0)),
            pl.BlockSpec((D, D), lambda i: (0, 0)),
            pl.BlockSpec((1, D), lambda i: (0, 0)),
        ],
        out_specs=pl.BlockSpec((ROWS_BLK, D), lambda i: (i, 0)),
        out_shape=jax.ShapeDtypeStruct((N_NODES, D), jnp.float32),
    )(x, s, w1, w2, b)


@jax.jit
def kernel(x, edge_index, W, b):
    src = edge_index[0]
    dst = edge_index[1]
    s_flat = _segmax_sc(src, dst, x)
    s = s_flat.reshape(NPAD, D)[:N_NODES]
    w1 = W[:D]
    w2 = W[D:]
    return _mlp_tc(x, s, w1, w2, b.reshape(1, D))


# speedup vs baseline: 2.6848x; 2.6848x over previous
import functools

import jax
import jax.numpy as jnp
from jax import lax
from jax.experimental import pallas as pl
from jax.experimental.pallas import tpu as pltpu
from jax.experimental.pallas import tpu_sc as plsc

N_NODES = 10000
N_EDGES = 320000
D = 128

NW = 32
NPT = 320
NPAD = NW * NPT
E_CHUNK = 4000
NCHUNK = N_EDGES // E_CHUNK
K = 128
R = 2
CAP = 22000
LIST = CAP + K + 16
NEG_INF = float("-inf")

_mesh = plsc.VectorSubcoreMesh(core_axis_name="c", subcore_axis_name="s")


def _lane_gather(v, idx):
    dnums = lax.GatherDimensionNumbers(
        offset_dims=(), collapsed_slice_dims=(0,), start_index_map=(0,))
    return lax.gather(v, idx[:, None], dnums, slice_sizes=(1,),
                      mode=lax.GatherScatterMode.PROMISE_IN_BOUNDS)


@functools.partial(
    pl.kernel,
    out_type=jax.ShapeDtypeStruct((NPAD * D,), jnp.float32),
    mesh=_mesh,
    compiler_params=pltpu.CompilerParams(needs_layout_passes=False),
    scratch_types=[
        pltpu.VMEM(((NPT + 1) * D,), jnp.float32),
        pltpu.VMEM((E_CHUNK,), jnp.int32),
        pltpu.VMEM((E_CHUNK,), jnp.int32),
        pltpu.VMEM((LIST,), jnp.int32),
        pltpu.VMEM((LIST,), jnp.int32),
        pltpu.VMEM((K, D), jnp.float32),
        pltpu.VMEM((K, D), jnp.float32),
        pltpu.SemaphoreType.DMA,
        pltpu.SemaphoreType.DMA,
    ],
)
def _segmax_sc(src_hbm, dst_hbm, x_hbm, out_hbm,
               acc, sr_v, ds_v, msrc, mdst, rows0, rows1, sem0, sem1):
    wid = lax.axis_index("s") * 2 + lax.axis_index("c")
    lo = wid * NPT
    rows = (rows0, rows1)
    sems = (sem0, sem1)

    neg = jnp.full((16,), NEG_INF, jnp.float32)
    def init_body(i, _):
        acc[pl.ds(i * 16, 16)] = neg
        return 0
    lax.fori_loop(0, (NPT + 1) * D // 16, init_body, 0)

    zero16 = jnp.zeros((16,), jnp.int32)
    dummy16 = jnp.full((16,), NPT, jnp.int32)
    iota16 = lax.iota(jnp.int32, 16)
    DUMP = jnp.int32(LIST - 1)
    _shift_idx = {k: jnp.maximum(iota16 - k, 0) for k in (1, 2, 4, 8)}
    _shift_msk = {k: -(1 - lax.shift_right_logical(iota16 - k, 31))
                  for k in (1, 2, 4, 8)}

    def _fire(g, r):
        pltpu.async_copy(x_hbm.at[msrc.at[pl.ds(g * K, K)]], rows[r], sems[r])

    def _wait(g, r):
        pltpu.make_async_copy(
            x_hbm.at[msrc.at[pl.ds(g * K, K)]], rows[r], sems[r]).wait()

    def _accum(g, r, cnt):
        nloc = jnp.minimum(K, cnt - g * K)
        nsub = (nloc + 15) // 16
        rref = rows[r]

        def sub_body(j16, _):
            j0 = j16 * 16
            dlv = mdst[pl.ds(g * K + j0, 16)] * D
            for l in range(16):
                base = dlv[l]
                for f in range(D // 16):
                    sl = pl.ds(base + f * 16, 16)
                    acc[sl] = jnp.maximum(
                        acc[sl], rref[j0 + l, pl.ds(f * 16, 16)])
            return 0
        lax.fori_loop(0, nsub, sub_body, 0)

    def _drain(cnt):
        for t in range(K // 16):
            msrc[pl.ds(cnt + t * 16, 16)] = zero16
            mdst[pl.ds(cnt + t * 16, 16)] = dummy16
        ng = (cnt + (K - 1)) // K
        for r in range(R):
            @pl.when(r < ng)
            def _():
                _fire(r, r)

        def outer(gg, _):
            for r in range(R):
                g = gg * R + r

                @pl.when(g < ng)
                def _():
                    _wait(g, r)
                    _accum(g, r, cnt)

                    @pl.when(g + R < ng)
                    def _():
                        _fire(g + R, r)
            return 0
        lax.fori_loop(0, (ng + (R - 1)) // R, outer, 0)

    def chunk_body(c, cnt):
        base_e = c * E_CHUNK
        pltpu.sync_copy(src_hbm.at[pl.ds(base_e, E_CHUNK)], sr_v)
        pltpu.sync_copy(dst_hbm.at[pl.ds(base_e, E_CHUNK)], ds_v)

        def filt_body(i, cnt):
            d = ds_v[pl.ds(i * 16, 16)]
            s = sr_v[pl.ds(i * 16, 16)]
            dl = d - lo
            bad = lax.shift_right_logical(dl | (NPT - 1 - dl), 31)
            mi = 1 - bad
            msk = -mi
            ps = mi
            for k in (1, 2, 4, 8):
                sh = _lane_gather(ps, _shift_idx[k])
                ps = ps + (sh & _shift_msk[k])
            base = cnt + ps - 1
            pos = (base & msk) | (DUMP & ~msk)
            plsc.store_scatter(msrc, [pos], s)
            plsc.store_scatter(mdst, [pos], dl)
            return cnt + ps[15]
        cnt = lax.fori_loop(0, E_CHUNK // 16, filt_body, cnt)

        full = cnt > (CAP - E_CHUNK)

        @pl.when(full)
        def _():
            _drain(cnt)
        return jnp.where(full, 0, cnt)

    cnt = lax.fori_loop(0, NCHUNK, chunk_body, jnp.int32(0))

    @pl.when(cnt > 0)
    def _():
        _drain(cnt)

    pltpu.sync_copy(acc.at[pl.ds(0, NPT * D)], out_hbm.at[pl.ds(lo * D, NPT * D)])


def _mlp_tc_body(x_ref, s_ref, w1_ref, w2_ref, b_ref, o_ref):
    x = x_ref[...]
    s = s_ref[...]
    xj = jnp.where(s == NEG_INF, 0.0, s - x)
    h = (jnp.dot(x, w1_ref[...], preferred_element_type=jnp.float32)
         + jnp.dot(xj, w2_ref[...], preferred_element_type=jnp.float32)
         + b_ref[...])
    o_ref[...] = jnp.maximum(h, 0.0)


ROWS_BLK = 1000


def _mlp_tc(x, s, w1, w2, b):
    grid = (N_NODES // ROWS_BLK,)
    return pl.pallas_call(
        _mlp_tc_body,
        grid=grid,
        in_specs=[
            pl.BlockSpec((ROWS_BLK, D), lambda i: (i, 0)),
            pl.BlockSpec((ROWS_BLK, D), lambda i: (i, 0)),
            pl.BlockSpec((D, D), lambda i: (0, 0)),
            pl.BlockSpec((D, D), lambda i: (0, 0)),
            pl.BlockSpec((1, D), lambda i: (0, 0)),
        ],
        out_specs=pl.BlockSpec((ROWS_BLK, D), lambda i: (i, 0)),
        out_shape=jax.ShapeDtypeStruct((N_NODES, D), jnp.float32),
    )(x, s, w1, w2, b)


@jax.jit
def kernel(x, edge_index, W, b):
    src = edge_index[0]
    dst = edge_index[1]
    s_flat = _segmax_sc(src, dst, x)
    s = s_flat.reshape(NPAD, D)[:N_NODES]
    w1 = W[:D]
    w2 = W[D:]
    return _mlp_tc(x, s, w1, w2, b.reshape(1, D))


# speedup vs baseline: 4.3685x; 1.6271x over previous
import functools

import jax
import jax.numpy as jnp
from jax import lax
from jax.experimental import pallas as pl
from jax.experimental.pallas import tpu as pltpu
from jax.experimental.pallas import tpu_sc as plsc

N_NODES = 10000
N_EDGES = 320000
D = 128

NW = 32
NPT = 320
NPAD = NW * NPT
E_CHUNK = 4000
NCHUNK = N_EDGES // E_CHUNK
K = 128
R = 2
CAP = 22000
LIST = CAP + K + 16
NEG_INF = float("-inf")

_mesh = plsc.VectorSubcoreMesh(core_axis_name="c", subcore_axis_name="s")


def _lane_gather(v, idx):
    dnums = lax.GatherDimensionNumbers(
        offset_dims=(), collapsed_slice_dims=(0,), start_index_map=(0,))
    return lax.gather(v, idx[:, None], dnums, slice_sizes=(1,),
                      mode=lax.GatherScatterMode.PROMISE_IN_BOUNDS)


@functools.partial(
    pl.kernel,
    out_type=jax.ShapeDtypeStruct((NPAD * D,), jnp.float32),
    mesh=_mesh,
    compiler_params=pltpu.CompilerParams(needs_layout_passes=False),
    scratch_types=[
        pltpu.VMEM(((NPT + 1) * D,), jnp.float32),
        pltpu.VMEM((E_CHUNK,), jnp.int32),
        pltpu.VMEM((E_CHUNK,), jnp.int32),
        pltpu.VMEM((LIST,), jnp.int32),
        pltpu.VMEM((LIST,), jnp.int32),
        pltpu.VMEM((K, D), jnp.float32),
        pltpu.VMEM((K, D), jnp.float32),
        pltpu.SemaphoreType.DMA,
        pltpu.SemaphoreType.DMA,
    ],
)
def _segmax_sc(src_hbm, dst_hbm, x_hbm, out_hbm,
               acc, sr_v, ds_v, msrc, mdst, rows0, rows1, sem0, sem1):
    wid = lax.axis_index("s") * 2 + lax.axis_index("c")
    lo = wid * NPT
    rows = (rows0, rows1)
    sems = (sem0, sem1)

    neg = jnp.full((16,), NEG_INF, jnp.float32)
    def init_body(i, _):
        acc[pl.ds(i * 16, 16)] = neg
        return 0
    lax.fori_loop(0, (NPT + 1) * D // 16, init_body, 0)

    zero16 = jnp.zeros((16,), jnp.int32)
    dummy16 = jnp.full((16,), NPT, jnp.int32)
    iota16 = lax.iota(jnp.int32, 16)
    DUMP = jnp.int32(LIST - 1)
    _shift_idx = {k: jnp.maximum(iota16 - k, 0) for k in (1, 2, 4, 8)}
    _shift_msk = {k: -(1 - lax.shift_right_logical(iota16 - k, 31))
                  for k in (1, 2, 4, 8)}
    _idx15 = jnp.full((16,), 15, jnp.int32)

    def _fire(g, r):
        pltpu.async_copy(x_hbm.at[msrc.at[pl.ds(g * K, K)]], rows[r], sems[r])

    def _wait(g, r):
        pltpu.make_async_copy(
            x_hbm.at[msrc.at[pl.ds(g * K, K)]], rows[r], sems[r]).wait()

    def _accum(g, r, cnt):
        nloc = jnp.minimum(K, cnt - g * K)
        nsub = (nloc + 15) // 16
        rref = rows[r]

        def sub_body(j16, _):
            j0 = j16 * 16
            dlv = mdst[pl.ds(g * K + j0, 16)] * D
            for l in range(16):
                base = dlv[l]
                rv = [rref[j0 + l, pl.ds(f * 16, 16)] for f in range(D // 16)]
                av = [acc[pl.ds(base + f * 16, 16)] for f in range(D // 16)]
                for f in range(D // 16):
                    acc[pl.ds(base + f * 16, 16)] = jnp.maximum(av[f], rv[f])
            return 0
        lax.fori_loop(0, nsub, sub_body, 0)

    def _drain(cnt):
        for t in range(K // 16):
            msrc[pl.ds(cnt + t * 16, 16)] = zero16
            mdst[pl.ds(cnt + t * 16, 16)] = dummy16
        ng = (cnt + (K - 1)) // K
        for r in range(R):
            @pl.when(r < ng)
            def _():
                _fire(r, r)

        def outer(gg, _):
            for r in range(R):
                g = gg * R + r

                @pl.when(g < ng)
                def _():
                    _wait(g, r)
                    _accum(g, r, cnt)

                    @pl.when(g + R < ng)
                    def _():
                        _fire(g + R, r)
            return 0
        lax.fori_loop(0, (ng + (R - 1)) // R, outer, 0)

    def chunk_body(c, cnt_v):
        base_e = c * E_CHUNK
        pltpu.sync_copy(src_hbm.at[pl.ds(base_e, E_CHUNK)], sr_v)
        pltpu.sync_copy(dst_hbm.at[pl.ds(base_e, E_CHUNK)], ds_v)

        def filt_body(i, cnt_v):
            res = []
            for h in range(2):
                d = ds_v[pl.ds(i * 32 + h * 16, 16)]
                s = sr_v[pl.ds(i * 32 + h * 16, 16)]
                dl = d - lo
                bad = lax.shift_right_logical(dl | (NPT - 1 - dl), 31)
                mi = 1 - bad
                ps = mi
                for k in (1, 2, 4, 8):
                    sh = _lane_gather(ps, _shift_idx[k])
                    ps = ps + (sh & _shift_msk[k])
                res.append((s, dl, -mi, ps))
            for (s, dl, msk, ps) in res:
                base = cnt_v + ps - 1
                pos = (base & msk) | (DUMP & ~msk)
                plsc.store_scatter(msrc, [pos], s)
                plsc.store_scatter(mdst, [pos], dl)
                cnt_v = cnt_v + _lane_gather(ps, _idx15)
            return cnt_v
        cnt_v = lax.fori_loop(0, E_CHUNK // 32, filt_body, cnt_v)
        cnt = cnt_v[0]

        full = cnt > (CAP - E_CHUNK)

        @pl.when(full)
        def _():
            _drain(cnt)
        return cnt_v * (1 - full.astype(jnp.int32))

    cnt_v = lax.fori_loop(0, NCHUNK, chunk_body, jnp.zeros((16,), jnp.int32))
    cnt = cnt_v[0]

    @pl.when(cnt > 0)
    def _():
        _drain(cnt)

    pltpu.sync_copy(acc.at[pl.ds(0, NPT * D)], out_hbm.at[pl.ds(lo * D, NPT * D)])


def _mlp_tc_body(x_ref, s_ref, w1_ref, w2_ref, b_ref, o_ref):
    x = x_ref[...]
    s = s_ref[...]
    xj = jnp.where(s == NEG_INF, 0.0, s - x)
    h = (jnp.dot(x, w1_ref[...], preferred_element_type=jnp.float32)
         + jnp.dot(xj, w2_ref[...], preferred_element_type=jnp.float32)
         + b_ref[...])
    o_ref[...] = jnp.maximum(h, 0.0)


ROWS_BLK = 1000


def _mlp_tc(x, s, w1, w2, b):
    grid = (N_NODES // ROWS_BLK,)
    return pl.pallas_call(
        _mlp_tc_body,
        grid=grid,
        in_specs=[
            pl.BlockSpec((ROWS_BLK, D), lambda i: (i, 0)),
            pl.BlockSpec((ROWS_BLK, D), lambda i: (i, 0)),
            pl.BlockSpec((D, D), lambda i: (0, 0)),
            pl.BlockSpec((D, D), lambda i: (0, 0)),
            pl.BlockSpec((1, D), lambda i: (0, 0)),
        ],
        out_specs=pl.BlockSpec((ROWS_BLK, D), lambda i: (i, 0)),
        out_shape=jax.ShapeDtypeStruct((N_NODES, D), jnp.float32),
    )(x, s, w1, w2, b)


@jax.jit
def kernel(x, edge_index, W, b):
    src = edge_index[0]
    dst = edge_index[1]
    s_flat = _segmax_sc(src, dst, x)
    s = s_flat.reshape(NPAD, D)[:N_NODES]
    w1 = W[:D]
    w2 = W[D:]
    return _mlp_tc(x, s, w1, w2, b.reshape(1, D))


# speedup vs baseline: 5.4224x; 1.2412x over previous
import functools

import jax
import jax.numpy as jnp
from jax import lax
from jax.experimental import pallas as pl
from jax.experimental.pallas import tpu as pltpu
from jax.experimental.pallas import tpu_sc as plsc

N_NODES = 10000
N_EDGES = 320000
D = 128

NW = 32
NPT = 320
NPAD = NW * NPT
E_CHUNK = 4096
NCHUNK = N_EDGES // E_CHUNK
E_TAIL = N_EDGES - NCHUNK * E_CHUNK
K = 128
R = 2
CAP = 18000
LIST = CAP + K + 16
NEG_INF = float("-inf")

_mesh = plsc.VectorSubcoreMesh(core_axis_name="c", subcore_axis_name="s")


def _lane_gather(v, idx):
    dnums = lax.GatherDimensionNumbers(
        offset_dims=(), collapsed_slice_dims=(0,), start_index_map=(0,))
    return lax.gather(v, idx[:, None], dnums, slice_sizes=(1,),
                      mode=lax.GatherScatterMode.PROMISE_IN_BOUNDS)


@functools.partial(
    pl.kernel,
    out_type=jax.ShapeDtypeStruct((NPAD * D,), jnp.float32),
    mesh=_mesh,
    compiler_params=pltpu.CompilerParams(needs_layout_passes=False),
    scratch_types=[
        pltpu.VMEM(((NPT + 1) * D,), jnp.float32),
        pltpu.VMEM((2, E_CHUNK), jnp.int32),
        pltpu.VMEM((2, E_CHUNK), jnp.int32),
        pltpu.VMEM((LIST,), jnp.int32),
        pltpu.VMEM((LIST,), jnp.int32),
        pltpu.VMEM((K, D), jnp.float32),
        pltpu.VMEM((K, D), jnp.float32),
        pltpu.SemaphoreType.DMA,
        pltpu.SemaphoreType.DMA,
        pltpu.SemaphoreType.DMA,
        pltpu.SemaphoreType.DMA,
    ],
)
def _segmax_sc(ei_hbm, x_hbm, out_hbm,
               acc, ei0, ei1, msrc, mdst, rows0, rows1,
               sem0, sem1, csem0, csem1):
    wid = lax.axis_index("s") * 2 + lax.axis_index("c")
    lo = wid * NPT
    rows = (rows0, rows1)
    sems = (sem0, sem1)
    eis = (ei0, ei1)
    csems = (csem0, csem1)

    def _cfire(c, p):
        pltpu.async_copy(
            ei_hbm.at[:, pl.ds(c * E_CHUNK, E_CHUNK)], eis[p], csems[p])

    def _cwait(c, p):
        pltpu.make_async_copy(
            ei_hbm.at[:, pl.ds(c * E_CHUNK, E_CHUNK)], eis[p], csems[p]).wait()

    neg = jnp.full((16,), NEG_INF, jnp.float32)
    def init_body(i, _):
        acc[pl.ds(i * 16, 16)] = neg
        return 0
    lax.fori_loop(0, (NPT + 1) * D // 16, init_body, 0)

    zero16 = jnp.zeros((16,), jnp.int32)
    dummy16 = jnp.full((16,), NPT, jnp.int32)
    iota16 = lax.iota(jnp.int32, 16)
    DUMP = jnp.int32(LIST - 1)
    _shift_idx = {k: jnp.maximum(iota16 - k, 0) for k in (1, 2, 4, 8)}
    _shift_msk = {k: -(1 - lax.shift_right_logical(iota16 - k, 31))
                  for k in (1, 2, 4, 8)}
    _idx15 = jnp.full((16,), 15, jnp.int32)

    def _fire(g, r):
        pltpu.async_copy(x_hbm.at[msrc.at[pl.ds(g * K, K)]], rows[r], sems[r])

    def _wait(g, r):
        pltpu.make_async_copy(
            x_hbm.at[msrc.at[pl.ds(g * K, K)]], rows[r], sems[r]).wait()

    def _accum(g, r, cnt):
        nloc = jnp.minimum(K, cnt - g * K)
        nsub = (nloc + 15) // 16
        rref = rows[r]

        def sub_body(j16, _):
            j0 = j16 * 16
            dlv = mdst[pl.ds(g * K + j0, 16)] * D
            for l in range(16):
                base = dlv[l]
                rv = [rref[j0 + l, pl.ds(f * 16, 16)] for f in range(D // 16)]
                av = [acc[pl.ds(base + f * 16, 16)] for f in range(D // 16)]
                for f in range(D // 16):
                    acc[pl.ds(base + f * 16, 16)] = jnp.maximum(av[f], rv[f])
            return 0
        lax.fori_loop(0, nsub, sub_body, 0)

    def _drain(cnt):
        for t in range(K // 16):
            msrc[pl.ds(cnt + t * 16, 16)] = zero16
            mdst[pl.ds(cnt + t * 16, 16)] = dummy16
        ng = (cnt + (K - 1)) // K
        for r in range(R):
            @pl.when(r < ng)
            def _():
                _fire(r, r)

        def outer(gg, _):
            for r in range(R):
                g = gg * R + r

                @pl.when(g < ng)
                def _():
                    _wait(g, r)
                    _accum(g, r, cnt)

                    @pl.when(g + R < ng)
                    def _():
                        _fire(g + R, r)
            return 0
        lax.fori_loop(0, (ng + (R - 1)) // R, outer, 0)

    def _one_chunk(c, p, cnt_v, niter, use_dma):
        ei_v = eis[p]
        if use_dma:
            _cwait(c, p)

            @pl.when(c + 1 < NCHUNK)
            def _():
                _cfire(c + 1, 1 - p)

        def filt_body(i, cnt_v):
            res = []
            for h in range(2):
                d = ei_v[1, pl.ds(i * 32 + h * 16, 16)]
                s = ei_v[0, pl.ds(i * 32 + h * 16, 16)]
                dl = d - lo
                bad = lax.shift_right_logical(dl | (NPT - 1 - dl), 31)
                mi = 1 - bad
                ps = mi
                for k in (1, 2, 4, 8):
                    sh = _lane_gather(ps, _shift_idx[k])
                    ps = ps + (sh & _shift_msk[k])
                res.append((s, dl, -mi, ps))
            for (s, dl, msk, ps) in res:
                base = cnt_v + ps - 1
                pos = (base & msk) | (DUMP & ~msk)
                plsc.store_scatter(msrc, [pos], s)
                plsc.store_scatter(mdst, [pos], dl)
                cnt_v = cnt_v + _lane_gather(ps, _idx15)
            return cnt_v
        cnt_v = lax.fori_loop(0, niter, filt_body, cnt_v)
        cnt = cnt_v[0]

        full = cnt > (CAP - E_CHUNK)

        @pl.when(full)
        def _():
            _drain(cnt)
        return cnt_v * (1 - full.astype(jnp.int32))

    _cfire(0, 0)

    def chunk_pair(cp, cnt_v):
        cnt_v = _one_chunk(cp * 2, 0, cnt_v, E_CHUNK // 32, True)
        cnt_v = _one_chunk(cp * 2 + 1, 1, cnt_v, E_CHUNK // 32, True)
        return cnt_v

    cnt_v = lax.fori_loop(0, NCHUNK // 2, chunk_pair, jnp.zeros((16,), jnp.int32))

    pltpu.sync_copy(ei_hbm.at[:, pl.ds(NCHUNK * E_CHUNK, E_TAIL)],
                    ei0.at[:, pl.ds(0, E_TAIL)])
    cnt_v = _one_chunk(NCHUNK, 0, cnt_v, E_TAIL // 32, False)
    cnt = cnt_v[0]

    @pl.when(cnt > 0)
    def _():
        _drain(cnt)

    pltpu.sync_copy(acc.at[pl.ds(0, NPT * D)], out_hbm.at[pl.ds(lo * D, NPT * D)])


def _mlp_tc_body(x_ref, s_ref, w1_ref, w2_ref, b_ref, o_ref):
    x = x_ref[...]
    s = s_ref[...]
    xj = jnp.where(s == NEG_INF, 0.0, s - x)
    h = (jnp.dot(x, w1_ref[...], preferred_element_type=jnp.float32)
         + jnp.dot(xj, w2_ref[...], preferred_element_type=jnp.float32)
         + b_ref[...])
    o_ref[...] = jnp.maximum(h, 0.0)


ROWS_BLK = 1000


def _mlp_tc(x, s, w1, w2, b):
    grid = (N_NODES // ROWS_BLK,)
    return pl.pallas_call(
        _mlp_tc_body,
        grid=grid,
        in_specs=[
            pl.BlockSpec((ROWS_BLK, D), lambda i: (i, 0)),
            pl.BlockSpec((ROWS_BLK, D), lambda i: (i, 0)),
            pl.BlockSpec((D, D), lambda i: (0, 0)),
            pl.BlockSpec((D, D), lambda i: (0, 0)),
            pl.BlockSpec((1, D), lambda i: (0, 0)),
        ],
        out_specs=pl.BlockSpec((ROWS_BLK, D), lambda i: (i, 0)),
        out_shape=jax.ShapeDtypeStruct((N_NODES, D), jnp.float32),
    )(x, s, w1, w2, b)


@jax.jit
def kernel(x, edge_index, W, b):
    s_flat = _segmax_sc(edge_index, x)
    s = s_flat.reshape(NPAD, D)[:N_NODES]
    w1 = W[:D]
    w2 = W[D:]
    return _mlp_tc(x, s, w1, w2, b.reshape(1, D))
